# baseline (speedup 1.0000x reference)
import functools

import jax
import jax.numpy as jnp
from jax.experimental import pallas as pl
from jax.experimental.pallas import tpu as pltpu

_BM = 400


def _pass1_body(gso_ref, x_ref, xb_ref, w0_ref, b0_ref, gbf_ref, p0_ref, r0_ref):
    g = gso_ref[...].astype(jnp.bfloat16)
    gbf_ref[...] = g
    y0 = jnp.dot(g, x_ref[...], preferred_element_type=jnp.float32)
    y0b = y0.astype(jnp.bfloat16)
    w0 = w0_ref[...]
    p0_ref[...] = jnp.dot(y0b, w0[2], preferred_element_type=jnp.float32).astype(
        jnp.bfloat16
    )
    r0_ref[...] = (
        jnp.dot(xb_ref[...], w0[0] - w0[2], preferred_element_type=jnp.float32)
        + jnp.dot(y0b, w0[1], preferred_element_type=jnp.float32)
        + b0_ref[...]
    )


def _fused234_body(nblk, n_class, gbf_hbm, p0_v, r0_v, w1_v, b1_v, out_hbm,
                   hbf_v, q1_v, s1_v):
    n = nblk * _BM
    gspec = pl.BlockSpec(
        (_BM, n), lambda i: (i, 0), pipeline_mode=pl.Buffered(buffer_count=5)
    )

    def rows_of(idx):
        return pl.ds(pl.multiple_of(idx[0] * _BM, _BM), _BM)

    def step2(idx, gslab):
        rows = rows_of(idx)
        out0 = (
            2.0 * jnp.dot(gslab[...], p0_v[...], preferred_element_type=jnp.float32)
            + r0_v[rows, :]
        )
        hb = jnp.maximum(out0, 0.0).astype(jnp.bfloat16)
        hbf_v[rows, :] = hb
        w1 = w1_v[...]
        s1_v[rows, :] = (
            jnp.dot(hb, w1[0] - w1[2], preferred_element_type=jnp.float32)
            + b1_v[...]
        )

    pltpu.emit_pipeline(
        step2, grid=(nblk,), in_specs=[gspec], _explicit_indices=True
    )(gbf_hbm)

    def step3(idx, gslab):
        rows = rows_of(idx)
        y1 = jnp.dot(gslab[...], hbf_v[...], preferred_element_type=jnp.float32)
        y1b = y1.astype(jnp.bfloat16)
        w1 = w1_v[...]
        q1_v[rows, :] = jnp.dot(
            y1b, w1[2], preferred_element_type=jnp.float32
        ).astype(jnp.bfloat16)
        s1_v[rows, :] = s1_v[rows, :] + jnp.dot(
            y1b, w1[1], preferred_element_type=jnp.float32
        )

    pltpu.emit_pipeline(
        step3, grid=(nblk,), in_specs=[gspec], _explicit_indices=True
    )(gbf_hbm)

    def step4(idx, gslab, outblk):
        rows = rows_of(idx)
        logits = (
            2.0 * jnp.dot(gslab[...], q1_v[...], preferred_element_type=jnp.float32)
            + s1_v[rows, :]
        )
        mask = jax.lax.broadcasted_iota(jnp.int32, logits.shape, 1) < n_class
        ml = jnp.where(mask, logits, -jnp.inf)
        m = jnp.max(ml, axis=1, keepdims=True)
        e = jnp.where(mask, jnp.exp(ml - m), 0.0)
        lse = m + jnp.log(jnp.sum(e, axis=1, keepdims=True))
        outblk[...] = (logits - lse)[:, :n_class]

    pltpu.emit_pipeline(
        step4,
        grid=(nblk,),
        in_specs=[gspec],
        out_specs=[pl.BlockSpec((_BM, n_class), lambda i: (i, 0))],
        _explicit_indices=True,
    )(gbf_hbm, out_hbm)


def kernel(x, gso, W0, b0, W1, b1):
    n, d = x.shape
    _, _, h_dim = W0.shape
    c = W1.shape[2]
    cp = 128
    nblk = n // _BM

    xb16 = x.astype(jnp.bfloat16)
    w0b = W0.astype(jnp.bfloat16)
    w1b = jnp.zeros((W1.shape[0], h_dim, cp), jnp.bfloat16)
    w1b = w1b.at[:, :, :c].set(W1.astype(jnp.bfloat16))
    b0r = b0.reshape(1, h_dim)
    b1r = jnp.zeros((1, cp), jnp.float32).at[0, :c].set(b1)

    row_blk = lambda bs: pl.BlockSpec(bs, lambda i: (i, 0))
    full2 = lambda shape: pl.BlockSpec(shape, lambda i: (0, 0))

    gbf, p0, r0 = pl.pallas_call(
        _pass1_body,
        grid=(nblk,),
        in_specs=[
            row_blk((_BM, n)),
            full2((n, d)),
            row_blk((_BM, d)),
            pl.BlockSpec((W0.shape[0], d, h_dim), lambda i: (0, 0, 0)),
            full2((1, h_dim)),
        ],
        out_specs=[
            row_blk((_BM, n)),
            row_blk((_BM, h_dim)),
            row_blk((_BM, h_dim)),
        ],
        out_shape=[
            jax.ShapeDtypeStruct((n, n), jnp.bfloat16),
            jax.ShapeDtypeStruct((n, h_dim), jnp.bfloat16),
            jax.ShapeDtypeStruct((n, h_dim), jnp.float32),
        ],
    )(gso, xb16, xb16, w0b, b0r)

    vmem_in = pl.BlockSpec(memory_space=pltpu.MemorySpace.VMEM)
    out = pl.pallas_call(
        functools.partial(_fused234_body, nblk, c),
        in_specs=[
            pl.BlockSpec(memory_space=pltpu.MemorySpace.HBM),
            vmem_in,
            vmem_in,
            vmem_in,
            vmem_in,
        ],
        out_specs=pl.BlockSpec(memory_space=pltpu.MemorySpace.HBM),
        out_shape=jax.ShapeDtypeStruct((n, c), jnp.float32),
        scratch_shapes=[
            pltpu.VMEM((n, h_dim), jnp.bfloat16),
            pltpu.VMEM((n, cp), jnp.bfloat16),
            pltpu.VMEM((n, cp), jnp.float32),
        ],
        compiler_params=pltpu.CompilerParams(
            vmem_limit_bytes=60 * 1024 * 1024,
        ),
    )(gbf, p0, r0, w1b, b1r)

    return out


# speedup vs baseline: 1.0258x; 1.0258x over previous
import functools

import jax
import jax.numpy as jnp
from jax.experimental import pallas as pl
from jax.experimental.pallas import tpu as pltpu

_BM1 = 200
_BM = 400


def _fused_body(nblk1, nblk, n_class, gso_hbm, x_v, w0_v, b0_v, w1_v, b1_v,
                gbf_hbm, out_hbm, p0_v, p0b_v, r0_v, hbf_v, q1_v, s1_v):
    n = nblk * _BM

    def step1(idx, gslab, gbfblk):
        rows = pl.ds(pl.multiple_of(idx[0] * _BM1, _BM1), _BM1)
        g = gslab[...].astype(jnp.bfloat16)
        gbfblk[...] = g
        xb = x_v[...]
        y0 = jnp.dot(g, xb, preferred_element_type=jnp.float32)
        y0b = y0.astype(jnp.bfloat16)
        w0 = w0_v[...]
        p0_v[rows, :] = jnp.dot(y0b, w0[2], preferred_element_type=jnp.float32)
        r0_v[rows, :] = (
            jnp.dot(x_v[rows, :], w0[0] - w0[2], preferred_element_type=jnp.float32)
            + jnp.dot(y0b, w0[1], preferred_element_type=jnp.float32)
            + b0_v[...]
        )

    pltpu.emit_pipeline(
        step1,
        grid=(nblk1,),
        in_specs=[pl.BlockSpec((_BM1, n), lambda i: (i, 0))],
        out_specs=[pl.BlockSpec((_BM1, n), lambda i: (i, 0))],
        _explicit_indices=True,
    )(gso_hbm, gbf_hbm)

    p0b_v[...] = p0_v[...].astype(jnp.bfloat16)

    gspec = pl.BlockSpec(
        (_BM, n), lambda i: (i, 0), pipeline_mode=pl.Buffered(buffer_count=4)
    )

    def rows_of(idx):
        return pl.ds(pl.multiple_of(idx[0] * _BM, _BM), _BM)

    def step2(idx, gslab):
        rows = rows_of(idx)
        out0 = (
            2.0 * jnp.dot(gslab[...], p0b_v[...], preferred_element_type=jnp.float32)
            + r0_v[rows, :]
        )
        hb = jnp.maximum(out0, 0.0).astype(jnp.bfloat16)
        hbf_v[rows, :] = hb
        w1 = w1_v[...]
        s1_v[rows, :] = (
            jnp.dot(hb, w1[0] - w1[2], preferred_element_type=jnp.float32)
            + b1_v[...]
        )

    pltpu.emit_pipeline(
        step2, grid=(nblk,), in_specs=[gspec], _explicit_indices=True
    )(gbf_hbm)

    def step3(idx, gslab):
        rows = rows_of(idx)
        y1 = jnp.dot(gslab[...], hbf_v[...], preferred_element_type=jnp.float32)
        y1b = y1.astype(jnp.bfloat16)
        w1 = w1_v[...]
        q1_v[rows, :] = jnp.dot(
            y1b, w1[2], preferred_element_type=jnp.float32
        ).astype(jnp.bfloat16)
        s1_v[rows, :] = s1_v[rows, :] + jnp.dot(
            y1b, w1[1], preferred_element_type=jnp.float32
        )

    pltpu.emit_pipeline(
        step3, grid=(nblk,), in_specs=[gspec], _explicit_indices=True
    )(gbf_hbm)

    def step4(idx, gslab, outblk):
        rows = rows_of(idx)
        logits = (
            2.0 * jnp.dot(gslab[...], q1_v[...], preferred_element_type=jnp.float32)
            + s1_v[rows, :]
        )
        mask = jax.lax.broadcasted_iota(jnp.int32, logits.shape, 1) < n_class
        ml = jnp.where(mask, logits, -jnp.inf)
        m = jnp.max(ml, axis=1, keepdims=True)
        e = jnp.where(mask, jnp.exp(ml - m), 0.0)
        lse = m + jnp.log(jnp.sum(e, axis=1, keepdims=True))
        outblk[...] = (logits - lse)[:, :n_class]

    pltpu.emit_pipeline(
        step4,
        grid=(nblk,),
        in_specs=[gspec],
        out_specs=[pl.BlockSpec((_BM, n_class), lambda i: (i, 0))],
        _explicit_indices=True,
    )(gbf_hbm, out_hbm)


def kernel(x, gso, W0, b0, W1, b1):
    n, d = x.shape
    _, _, h_dim = W0.shape
    c = W1.shape[2]
    cp = 128

    xb16 = x.astype(jnp.bfloat16)
    w0b = W0.astype(jnp.bfloat16)
    w1b = jnp.zeros((W1.shape[0], h_dim, cp), jnp.bfloat16)
    w1b = w1b.at[:, :, :c].set(W1.astype(jnp.bfloat16))
    b0r = b0.reshape(1, h_dim)
    b1r = jnp.zeros((1, cp), jnp.float32).at[0, :c].set(b1)

    vmem_in = pl.BlockSpec(memory_space=pltpu.MemorySpace.VMEM)
    hbm = pl.BlockSpec(memory_space=pltpu.MemorySpace.HBM)
    _, out = pl.pallas_call(
        functools.partial(_fused_body, n // _BM1, n // _BM, c),
        in_specs=[
            hbm,
            vmem_in,
            vmem_in,
            vmem_in,
            vmem_in,
            vmem_in,
        ],
        out_specs=[hbm, hbm],
        out_shape=[
            jax.ShapeDtypeStruct((n, n), jnp.bfloat16),
            jax.ShapeDtypeStruct((n, c), jnp.float32),
        ],
        scratch_shapes=[
            pltpu.VMEM((n, h_dim), jnp.float32),
            pltpu.VMEM((n, h_dim), jnp.bfloat16),
            pltpu.VMEM((n, h_dim), jnp.float32),
            pltpu.VMEM((n, h_dim), jnp.bfloat16),
            pltpu.VMEM((n, cp), jnp.bfloat16),
            pltpu.VMEM((n, cp), jnp.float32),
        ],
        compiler_params=pltpu.CompilerParams(
            vmem_limit_bytes=60 * 1024 * 1024,
        ),
    )(gso, xb16, w0b, b0r, w1b, b1r)

    return out


# speedup vs baseline: 1.0286x; 1.0028x over previous
import functools

import jax
import jax.numpy as jnp
from jax.experimental import pallas as pl
from jax.experimental.pallas import tpu as pltpu

_BM1 = 200
_BM = 400


def _fused_body(nblk1, nblk, n_class, gso_hbm, x_v, w0_v, b0_v, w1_v, b1_v,
                gbf_hbm, out_hbm, p0_v, p0b_v, r0_v, hbf_v, q1_v, s1_v):
    n = nblk * _BM

    def step1(idx, gslab, gbfblk):
        rows = pl.ds(pl.multiple_of(idx[0] * _BM1, _BM1), _BM1)
        g = gslab[...].astype(jnp.bfloat16)
        gbfblk[...] = g
        xb = x_v[...]
        y0 = jnp.dot(g, xb, preferred_element_type=jnp.float32)
        y0b = y0.astype(jnp.bfloat16)
        w0 = w0_v[...]
        p0_v[rows, :] = jnp.dot(y0b, w0[2], preferred_element_type=jnp.float32)
        r0_v[rows, :] = (
            jnp.dot(x_v[rows, :], w0[0] - w0[2], preferred_element_type=jnp.float32)
            + jnp.dot(y0b, w0[1], preferred_element_type=jnp.float32)
            + b0_v[...]
        )

    pltpu.emit_pipeline(
        step1,
        grid=(nblk1,),
        in_specs=[pl.BlockSpec((_BM1, n), lambda i: (i, 0),
                               pipeline_mode=pl.Buffered(buffer_count=3))],
        out_specs=[pl.BlockSpec((_BM1, n), lambda i: (i, 0))],
        _explicit_indices=True,
    )(gso_hbm, gbf_hbm)

    p0b_v[...] = p0_v[...].astype(jnp.bfloat16)

    gspec = pl.BlockSpec(
        (_BM, n), lambda i: (i, 0), pipeline_mode=pl.Buffered(buffer_count=4)
    )

    def rows_of(idx):
        return pl.ds(pl.multiple_of(idx[0] * _BM, _BM), _BM)

    def step2(idx, gslab):
        rows = rows_of(idx)
        out0 = (
            2.0 * jnp.dot(gslab[...], p0b_v[...], preferred_element_type=jnp.float32)
            + r0_v[rows, :]
        )
        hb = jnp.maximum(out0, 0.0).astype(jnp.bfloat16)
        hbf_v[rows, :] = hb
        w1 = w1_v[...]
        s1_v[rows, :] = (
            jnp.dot(hb, w1[0] - w1[2], preferred_element_type=jnp.float32)
            + b1_v[...]
        )

    pltpu.emit_pipeline(
        step2, grid=(nblk,), in_specs=[gspec], _explicit_indices=True
    )(gbf_hbm)

    def step3(idx, gslab):
        rows = rows_of(idx)
        y1 = jnp.dot(gslab[...], hbf_v[...], preferred_element_type=jnp.float32)
        y1b = y1.astype(jnp.bfloat16)
        w1 = w1_v[...]
        q1_v[rows, :] = jnp.dot(
            y1b, w1[2], preferred_element_type=jnp.float32
        ).astype(jnp.bfloat16)
        s1_v[rows, :] = s1_v[rows, :] + jnp.dot(
            y1b, w1[1], preferred_element_type=jnp.float32
        )

    pltpu.emit_pipeline(
        step3, grid=(nblk,), in_specs=[gspec], _explicit_indices=True
    )(gbf_hbm)

    def step4(idx, gslab, outblk):
        rows = rows_of(idx)
        logits = (
            2.0 * jnp.dot(gslab[...], q1_v[...], preferred_element_type=jnp.float32)
            + s1_v[rows, :]
        )
        mask = jax.lax.broadcasted_iota(jnp.int32, logits.shape, 1) < n_class
        ml = jnp.where(mask, logits, -jnp.inf)
        m = jnp.max(ml, axis=1, keepdims=True)
        e = jnp.where(mask, jnp.exp(ml - m), 0.0)
        lse = m + jnp.log(jnp.sum(e, axis=1, keepdims=True))
        outblk[...] = (logits - lse)[:, :n_class]

    pltpu.emit_pipeline(
        step4,
        grid=(nblk,),
        in_specs=[gspec],
        out_specs=[pl.BlockSpec((_BM, n_class), lambda i: (i, 0))],
        _explicit_indices=True,
    )(gbf_hbm, out_hbm)


def kernel(x, gso, W0, b0, W1, b1):
    n, d = x.shape
    _, _, h_dim = W0.shape
    c = W1.shape[2]
    cp = 128

    xb16 = x.astype(jnp.bfloat16)
    w0b = W0.astype(jnp.bfloat16)
    w1b = jnp.zeros((W1.shape[0], h_dim, cp), jnp.bfloat16)
    w1b = w1b.at[:, :, :c].set(W1.astype(jnp.bfloat16))
    b0r = b0.reshape(1, h_dim)
    b1r = jnp.zeros((1, cp), jnp.float32).at[0, :c].set(b1)

    vmem_in = pl.BlockSpec(memory_space=pltpu.MemorySpace.VMEM)
    hbm = pl.BlockSpec(memory_space=pltpu.MemorySpace.HBM)
    _, out = pl.pallas_call(
        functools.partial(_fused_body, n // _BM1, n // _BM, c),
        in_specs=[
            hbm,
            vmem_in,
            vmem_in,
            vmem_in,
            vmem_in,
            vmem_in,
        ],
        out_specs=[hbm, hbm],
        out_shape=[
            jax.ShapeDtypeStruct((n, n), jnp.bfloat16),
            jax.ShapeDtypeStruct((n, c), jnp.float32),
        ],
        scratch_shapes=[
            pltpu.VMEM((n, h_dim), jnp.float32),
            pltpu.VMEM((n, h_dim), jnp.bfloat16),
            pltpu.VMEM((n, h_dim), jnp.float32),
            pltpu.VMEM((n, h_dim), jnp.bfloat16),
            pltpu.VMEM((n, cp), jnp.bfloat16),
            pltpu.VMEM((n, cp), jnp.float32),
        ],
        compiler_params=pltpu.CompilerParams(
            vmem_limit_bytes=60 * 1024 * 1024,
        ),
    )(gso, xb16, w0b, b0r, w1b, b1r)

    return out
